# Initial kernel scaffold; baseline (speedup 1.0000x reference)
import jax
import jax.numpy as jnp
from jax.experimental import pallas as pl


def kernel(h_src, x_src, edges_src, edge_attr_src, h_tgt, x_tgt, edges_tgt, edge_attr_tgt, corr, labels, W1, b1, W2, b2, W3, b3, W4, b4):
    raise NotImplementedError("write your pallas kernel here")


# speedup vs baseline: 2.1531x; 2.1531x over previous
import functools

import jax
import jax.numpy as jnp
from jax import lax
from jax.experimental import pallas as pl
from jax.experimental.pallas import tpu as pltpu
from jax.experimental.pallas import tpu_sc as plsc

_B, _N, _D = 32, 8192, 32
_K = 128
_XW = 8
_FMIN = -3.4028235e38


def _topk_gather(sim_flat, xcat_flat):
    info = plsc.get_sparse_core_info()
    nc, ns = info.num_cores, info.num_subcores
    nw = nc * ns
    per_w = _B // nw if nw <= _B else 1
    mesh = plsc.VectorSubcoreMesh(core_axis_name="c", subcore_axis_name="s")
    nchunk = _N // 16
    nvec = nchunk // 16

    @functools.partial(
        pl.kernel,
        out_type=jax.ShapeDtypeStruct((_B * _K, _XW), jnp.float32),
        mesh=mesh,
        compiler_params=pltpu.CompilerParams(needs_layout_passes=False,
                                             use_tc_tiling_on_sc=False),
        scratch_types=[
            pltpu.VMEM((_N,), jnp.float32),
            pltpu.VMEM((nchunk,), jnp.float32),
            pltpu.VMEM((nvec * 16,), jnp.float32),
            pltpu.VMEM((_K,), jnp.int32),
            pltpu.VMEM((_K, _XW), jnp.float32),
            pltpu.SemaphoreType.DMA,
        ],
    )
    def k(sim_hbm, xcat_hbm, out_hbm, sim_v, cm_v, scm_v, idx_v, rows_v, sem):
        wid = lax.axis_index("s") * nc + lax.axis_index("c")
        lanes = lax.iota(jnp.int32, 16)
        big = jnp.int32(9999)
        lane0 = lanes == 0

        def store1(ref, pos, val):
            plsc.store_scatter(ref, [jnp.full((16,), pos, jnp.int32)],
                               jnp.full((16,), val), mask=lane0)

        for b0 in range(per_w):
            b = wid * per_w + b0
            pltpu.sync_copy(sim_hbm.at[pl.ds(b * _N, _N)], sim_v)

            def cm_body(i, carry):
                store1(cm_v, i, jnp.max(sim_v[pl.ds(i * 16, 16)]))
                return carry

            lax.fori_loop(0, nchunk, cm_body, 0)

            def scm_body(j, carry):
                store1(scm_v, j, jnp.max(cm_v[pl.ds(j * 16, 16)]))
                return carry

            lax.fori_loop(0, nvec, scm_body, 0)

            def ext_body(ki, carry):
                s0 = scm_v[pl.ds(0, 16)]
                s1 = scm_v[pl.ds(16, 16)]
                m = jnp.maximum(jnp.max(s0), jnp.max(s1))
                j0 = jnp.min(jnp.where(s0 == m, lanes, big))
                j1 = jnp.min(jnp.where(s1 == m, lanes, big)) + 16
                j = jnp.minimum(j0, j1)
                cmv = cm_v[pl.ds(j * 16, 16)]
                l2 = jnp.min(jnp.where(cmv == m, lanes, big))
                chunk = j * 16 + l2
                dv = sim_v[pl.ds(chunk * 16, 16)]
                lane = jnp.min(jnp.where(dv == m, lanes, big))
                store1(idx_v, ki, chunk * 16 + lane + b * _N)
                dv2 = jnp.where(lanes == lane, jnp.float32(_FMIN), dv)
                sim_v[pl.ds(chunk * 16, 16)] = dv2
                nm = jnp.max(dv2)
                store1(cm_v, chunk, nm)
                store1(scm_v, j, jnp.max(jnp.where(lanes == l2, nm, cmv)))
                return carry

            lax.fori_loop(0, _K, ext_body, 0)

            pltpu.async_copy(xcat_hbm.at[idx_v], rows_v, sem).wait()
            pltpu.sync_copy(rows_v, out_hbm.at[pl.ds(b * _K, _K)])

    return k(sim_flat, xcat_flat)


def _mlp_body(x_ref, w1_ref, b1_ref, w2_ref, b2_ref, w3_ref, b3_ref,
              w4_ref, b4_ref, out_ref):
    def mm(a, w):
        return jnp.dot(a.astype(jnp.bfloat16), w.astype(jnp.bfloat16),
                       preferred_element_type=jnp.float32)

    h = x_ref[...]
    h = jnp.maximum(mm(h, w1_ref[...]) + b1_ref[...], 0.0)
    h = jnp.maximum(mm(h, w2_ref[...]) + b2_ref[...], 0.0)
    h = jnp.maximum(mm(h, w3_ref[...]) + b3_ref[...], 0.0)
    out_ref[...] = mm(h, w4_ref[...]) + b4_ref[...]


def _mlp_call(flat, W1, b1, W2, b2, W3, b3, W4, b4):
    return pl.pallas_call(
        _mlp_body,
        out_shape=jax.ShapeDtypeStruct((_B, 7), jnp.float32),
    )(flat, W1, b1.reshape(1, -1), W2, b2.reshape(1, -1),
      W3, b3.reshape(1, -1), W4, b4.reshape(1, -1))


def _normalize(v, axis=-1):
    n = jnp.sqrt(jnp.sum(v * v, axis=axis, keepdims=True))
    return v / jnp.maximum(n, 1e-12)


def _quat_to_mat(q):
    q = _normalize(q)
    qx, qy, qz, qw = q[..., 0], q[..., 1], q[..., 2], q[..., 3]
    r00 = 1 - 2 * (qy ** 2 + qz ** 2)
    r01 = 2 * (qx * qy - qz * qw)
    r02 = 2 * (qx * qz + qy * qw)
    r10 = 2 * (qx * qy + qz * qw)
    r11 = 1 - 2 * (qx ** 2 + qz ** 2)
    r12 = 2 * (qy * qz - qx * qw)
    r20 = 2 * (qx * qz - qy * qw)
    r21 = 2 * (qy * qz + qx * qw)
    r22 = 1 - 2 * (qx ** 2 + qy ** 2)
    row0 = jnp.stack([r00, r01, r02], -1)
    row1 = jnp.stack([r10, r11, r12], -1)
    row2 = jnp.stack([r20, r21, r22], -1)
    return jnp.stack([row0, row1, row2], -2)


def _pose_to_quat(Rm):
    tr = jnp.trace(Rm, axis1=1, axis2=2)
    r = lambda i, j: Rm[:, i, j]
    eps = 1e-9
    S0 = jnp.sqrt(jnp.maximum(tr + 1.0, eps)) * 2.0
    q0 = jnp.stack([0.25 * S0, (r(2, 1) - r(1, 2)) / S0, (r(0, 2) - r(2, 0)) / S0, (r(1, 0) - r(0, 1)) / S0], -1)
    S1 = jnp.sqrt(jnp.maximum(1.0 + r(0, 0) - r(1, 1) - r(2, 2), eps)) * 2.0
    q1 = jnp.stack([(r(2, 1) - r(1, 2)) / S1, 0.25 * S1, (r(0, 1) + r(1, 0)) / S1, (r(0, 2) + r(2, 0)) / S1], -1)
    S2 = jnp.sqrt(jnp.maximum(1.0 + r(1, 1) - r(0, 0) - r(2, 2), eps)) * 2.0
    q2 = jnp.stack([(r(0, 2) - r(2, 0)) / S2, (r(0, 1) + r(1, 0)) / S2, 0.25 * S2, (r(1, 2) + r(2, 1)) / S2], -1)
    S3 = jnp.sqrt(jnp.maximum(1.0 + r(2, 2) - r(0, 0) - r(1, 1), eps)) * 2.0
    q3 = jnp.stack([(r(1, 0) - r(0, 1)) / S3, (r(0, 2) + r(2, 0)) / S3, (r(1, 2) + r(2, 1)) / S3, 0.25 * S3], -1)
    c0 = (tr > 0.0)[:, None]
    c1 = ((r(0, 0) > r(1, 1)) & (r(0, 0) > r(2, 2)))[:, None]
    c2 = (r(1, 1) > r(2, 2))[:, None]
    return jnp.where(c0, q0, jnp.where(c1, q1, jnp.where(c2, q2, q3)))


def kernel(h_src, x_src, edges_src, edge_attr_src, h_tgt, x_tgt, edges_tgt,
           edge_attr_tgt, corr, labels, W1, b1, W2, b2, W3, b3, W4, b4):
    scores = jnp.sum(h_src * h_tgt, axis=-1)

    pad = jnp.zeros((_B, _N, _XW - 6), jnp.float32)
    xcat = jnp.concatenate([x_src, x_tgt, pad], axis=-1).reshape(_B * _N, _XW)
    cx = _topk_gather(scores.reshape(_B * _N), xcat).reshape(_B, _K, _XW)

    mask = labels[..., 0] != 0
    maskf = mask.astype(x_src.dtype)
    ms = jnp.where(mask, scores, -1e30)
    w = jax.nn.softmax(ms, axis=-1) * maskf
    has = jnp.any(mask, axis=-1)
    src_c = jnp.sum(w[..., None] * x_src, axis=1)
    tgt_c = jnp.sum(w[..., None] * x_tgt, axis=1)
    sc = x_src - src_c[:, None, :]
    tc = x_tgt - tgt_c[:, None, :]
    H = jnp.einsum('bn,bni,bnj->bij', w, sc, tc)
    eye = jnp.broadcast_to(jnp.eye(3, dtype=x_src.dtype), H.shape)
    Hs = jnp.where(has[:, None, None], H, eye)
    U, S, Vt = jnp.linalg.svd(Hs)
    R1 = jnp.einsum('bji,bkj->bik', Vt, U)
    det = jnp.linalg.det(R1)
    sign = jnp.where(det < 0, -1.0, 1.0)
    factor = jnp.concatenate([jnp.ones((sign.shape[0], 2), dtype=Vt.dtype), sign[:, None]], -1)[..., None]
    Vt2 = Vt * factor
    Rk = jnp.einsum('bji,bkj->bik', Vt2, U)
    Rk = jnp.where(has[:, None, None], Rk, eye)
    t = tgt_c - jnp.einsum('bij,bj->bi', Rk, src_c)
    t = jnp.where(has[:, None], t, 0.0)

    flat = cx[:, :, 0:6].reshape(_B, _K * 6)
    delta = _mlp_call(flat, W1, b1, W2, b2, W3, b3, W4, b4)
    dq = _normalize(delta[:, :4])
    dt = delta[:, 4:]
    dR = _quat_to_mat(dq)
    rR = jnp.einsum('bij,bjk->bik', dR, Rk)
    rt = t + dt
    top = jnp.concatenate([rR, rt[:, :, None]], -1)
    bottom = jnp.broadcast_to(jnp.array([0.0, 0.0, 0.0, 1.0], dtype=rR.dtype), (_B, 1, 4))
    pose = jnp.concatenate([top, bottom], 1)
    quat = _normalize(_pose_to_quat(pose))
    trans = pose[:, :3, 3]
    return (quat, trans, h_src, x_src, h_tgt, x_tgt, labels)
